# baseline (speedup 1.0000x reference)
import functools

import jax
import jax.numpy as jnp
from jax import lax
from jax.experimental import pallas as pl
from jax.experimental.pallas import tpu as pltpu
from jax.experimental.pallas import tpu_sc as plsc

N = 10000
NP = 10240
E = 320000
EP = 327680
F_IN = 128
H2 = 256
HALF = 128
C = 2
G = 128

NC = 2
NS = 16
CHUNK = 128
ECH = EP // CHUNK
SPC = 32
ECHS = EP // SPC

NCH_DEG = ECH // (NC * NS)
NCH_SP = ECHS // NS
NSTG = 16
SCH = NCH_SP // NSTG
NBUF = 8
KPF = 4
RPT = NP // NS
CPT = NP // NS

f32 = jnp.float32


@functools.lru_cache(maxsize=None)
def _sc_kernels():
    mesh = plsc.VectorSubcoreMesh(
        core_axis_name="c", subcore_axis_name="s",
        num_cores=NC, num_subcores=NS)

    @functools.partial(
        pl.kernel,
        out_type=jax.ShapeDtypeStruct((NC, 1, NP), f32),
        mesh=mesh,
        scratch_types=[
            pltpu.VMEM((NCH_DEG, CHUNK), jnp.int32),
            pltpu.VMEM((CHUNK,), f32),
            pltpu.VMEM((CPT,), f32),
            pltpu.VMEM_SHARED((NP,), f32),
        ],
    )
    def sc_deg1(dst_hbm, part_hbm, idx_v, ones_v, zbuf, acc_sh):
        c = lax.axis_index("c")
        s = lax.axis_index("s")
        wid = c * NS + s

        one16 = jnp.ones((16,), f32)
        zero16 = jnp.zeros((16,), f32)

        def fill_ones(i, _):
            ones_v[pl.ds(i * 16, 16)] = one16
            return 0

        lax.fori_loop(0, CHUNK // 16, fill_ones, 0)

        def fill_zero(i, _):
            zbuf[pl.ds(i * 16, 16)] = zero16
            return 0

        lax.fori_loop(0, CPT // 16, fill_zero, 0)

        pltpu.sync_copy(zbuf, acc_sh.at[pl.ds(s * CPT, CPT)])
        pltpu.sync_copy(dst_hbm.at[pl.ds(wid * NCH_DEG, NCH_DEG)], idx_v)
        plsc.subcore_barrier()

        def body(j, _):
            pltpu.sync_copy(ones_v, acc_sh.at[idx_v.at[j]], add=True)
            return 0

        lax.fori_loop(0, NCH_DEG, body, 0)
        plsc.subcore_barrier()
        pltpu.sync_copy(acc_sh.at[pl.ds(s * CPT, CPT)],
                        part_hbm.at[c, 0, pl.ds(s * CPT, CPT)])

    @functools.partial(
        pl.kernel,
        out_type=jax.ShapeDtypeStruct((NP,), f32),
        mesh=mesh,
        scratch_types=[
            pltpu.VMEM((NC, CPT), f32),
            pltpu.VMEM((CPT,), f32),
        ],
    )
    def sc_deg2(part_hbm, deg_hbm, pbuf, abuf):
        c = lax.axis_index("c")
        s = lax.axis_index("s")

        @pl.when(c == 0)
        def _():
            pltpu.sync_copy(
                part_hbm.at[:, 0, pl.ds(s * CPT, CPT)], pbuf)

            def red(k, _):
                v = pbuf[0, pl.ds(k * 16, 16)]
                for i in range(1, NC):
                    v = v + pbuf[i, pl.ds(k * 16, 16)]
                abuf[pl.ds(k * 16, 16)] = v
                return 0

            lax.fori_loop(0, CPT // 16, red, 0)
            pltpu.sync_copy(abuf, deg_hbm.at[pl.ds(s * CPT, CPT)])

    @functools.partial(
        pl.kernel,
        out_type=(
            jax.ShapeDtypeStruct((NP, HALF), f32),
            jax.ShapeDtypeStruct((NP, HALF), f32),
        ),
        mesh=mesh,
        scratch_types=(
            [pltpu.VMEM((SCH, SPC), jnp.int32)] * 2
            + [pltpu.VMEM((SPC, HALF), f32)] * NBUF
            + [pltpu.VMEM_SHARED((NP, HALF), f32)]
            + [pltpu.SemaphoreType.DMA] * (2 * NBUF)
        ),
    )
    def sc_spmm(hlo_hbm, hhi_hbm, src_hbm, dst_hbm, alo_hbm, ahi_hbm,
                *scratch):
        src_v, dst_v = scratch[0], scratch[1]
        rows = scratch[2:2 + NBUF]
        acc_sh = scratch[2 + NBUF]
        gsems = scratch[3 + NBUF:3 + 2 * NBUF]
        ssems = scratch[3 + 2 * NBUF:3 + 3 * NBUF]
        c = lax.axis_index("c")
        s = lax.axis_index("s")

        def do_half(h_ref, agg_ref):
            pltpu.sync_copy(
                h_ref.at[pl.ds(s * RPT, RPT)],
                acc_sh.at[pl.ds(s * RPT, RPT)],
            )
            plsc.subcore_barrier()

            def gather(jj, b):
                pltpu.async_copy(h_ref.at[src_v.at[jj]], rows[b], gsems[b])

            def gwait(b):
                pltpu.make_async_copy(
                    h_ref.at[pl.ds(0, SPC)], rows[b], gsems[b]).wait()

            def scatter(jj, b):
                pltpu.async_copy(
                    rows[b], acc_sh.at[dst_v.at[jj]], ssems[b], add=True)

            def swait(b):
                pltpu.make_async_copy(
                    rows[b], acc_sh.at[pl.ds(0, SPC)], ssems[b]).wait()

            def stage(q):
                base = pl.multiple_of(s * NCH_SP + q * SCH, 8)
                pltpu.sync_copy(src_hbm.at[pl.ds(base, SCH)], src_v)
                pltpu.sync_copy(dst_hbm.at[pl.ds(base, SCH)], dst_v)
                for t in range(KPF):
                    gather(t, t)

                def lbody(j):
                    for bi in range(NBUF):
                        jj = j + bi
                        b = bi
                        nb = (bi + KPF) % NBUF

                        @pl.when(jj >= NBUF - KPF)
                        def _():
                            swait(nb)

                        @pl.when(jj + KPF < SCH)
                        def _():
                            gather(jj + KPF, nb)

                        gwait(b)
                        scatter(jj, b)

                pl.loop(0, SCH, step=NBUF)(lbody)
                for m in range(SCH - (NBUF - KPF), SCH):
                    swait(m % NBUF)

            pl.loop(0, NSTG)(stage)
            plsc.subcore_barrier()

            pltpu.sync_copy(
                acc_sh.at[pl.ds(s * RPT, RPT)],
                agg_ref.at[pl.ds(s * RPT, RPT)],
            )
            plsc.subcore_barrier()

        @pl.when(c == 0)
        def _():
            do_half(hlo_hbm, alo_hbm)

        @pl.when(c == 1)
        def _():
            do_half(hhi_hbm, ahi_hbm)

    return sc_deg1, sc_deg2, sc_spmm


_SELU_ALPHA = 1.6732632423543772
_SELU_SCALE = 1.0507009873554805


def _selu(x):
    return _SELU_SCALE * jnp.where(x > 0, x, _SELU_ALPHA * (jnp.exp(x) - 1.0))


RB = 2560
NSTEP = NP // RB


def _tc_pre_body(deg_ref, x_ref, w1_ref, batch_ref,
                 hlo_ref, hhi_ref, dinv_ref, fidx_ref, cnt_acc):
    j = pl.program_id(0)

    @pl.when(j == 0)
    def _():
        cnt_acc[...] = jnp.zeros_like(cnt_acc)

    deg = deg_ref[...] + 1.0
    dinv = lax.rsqrt(deg)
    dinv_ref[...] = dinv
    h = jnp.dot(x_ref[...], w1_ref[...], preferred_element_type=f32) * dinv
    hlo_ref[...] = h[:, :HALF]
    hhi_ref[...] = h[:, HALF:]

    gvec = lax.broadcasted_iota(jnp.int32, (1, G), 1)
    lt = (batch_ref[...] < gvec).astype(f32)
    cnt_acc[...] += jnp.sum(lt, axis=0, keepdims=True)

    @pl.when(j == NSTEP - 1)
    def _():
        fidx_ref[...] = jnp.minimum(cnt_acc[...], float(N - 1))


def _tc_pre(deg, x, w1, batch2):
    return pl.pallas_call(
        _tc_pre_body,
        grid=(NSTEP,),
        in_specs=[
            pl.BlockSpec((RB, 1), lambda j: (j, 0)),
            pl.BlockSpec((RB, F_IN), lambda j: (j, 0)),
            pl.BlockSpec((F_IN, H2), lambda j: (0, 0)),
            pl.BlockSpec((RB, 1), lambda j: (j, 0)),
        ],
        out_specs=[
            pl.BlockSpec((RB, HALF), lambda j: (j, 0)),
            pl.BlockSpec((RB, HALF), lambda j: (j, 0)),
            pl.BlockSpec((RB, 1), lambda j: (j, 0)),
            pl.BlockSpec((1, G), lambda j: (0, 0)),
        ],
        out_shape=[
            jax.ShapeDtypeStruct((NP, HALF), f32),
            jax.ShapeDtypeStruct((NP, HALF), f32),
            jax.ShapeDtypeStruct((NP, 1), f32),
            jax.ShapeDtypeStruct((1, G), f32),
        ],
        scratch_shapes=[pltpu.VMEM((1, G), f32)],
    )(deg, x, w1, batch2)


def _tc_mid_body(alo_ref, ahi_ref, dinv_ref, b1_ref, w2_ref,
                 hlo_ref, hhi_ref):
    dinv = dinv_ref[...]
    a = jnp.concatenate([alo_ref[...], ahi_ref[...]], axis=1)
    h1 = _selu(a * dinv + b1_ref[...])
    h2 = jnp.dot(h1, w2_ref[...], preferred_element_type=f32) * dinv
    hlo_ref[...] = h2[:, :HALF]
    hhi_ref[...] = h2[:, HALF:]


def _tc_mid(alo, ahi, dinv, b1, w2):
    return pl.pallas_call(
        _tc_mid_body,
        grid=(NSTEP,),
        in_specs=[
            pl.BlockSpec((RB, HALF), lambda j: (j, 0)),
            pl.BlockSpec((RB, HALF), lambda j: (j, 0)),
            pl.BlockSpec((RB, 1), lambda j: (j, 0)),
            pl.BlockSpec((1, H2), lambda j: (0, 0)),
            pl.BlockSpec((H2, H2), lambda j: (0, 0)),
        ],
        out_specs=[
            pl.BlockSpec((RB, HALF), lambda j: (j, 0)),
            pl.BlockSpec((RB, HALF), lambda j: (j, 0)),
        ],
        out_shape=[
            jax.ShapeDtypeStruct((NP, HALF), f32),
            jax.ShapeDtypeStruct((NP, HALF), f32),
        ],
    )(alo, ahi, dinv, b1, w2)


def _tc_head_body(alo_ref, ahi_ref, dinv_ref, b2_ref, batch_ref, x_ref,
                  fidx_ref, wf0_ref, bf0_ref, wf1_ref, bf1_ref, wf2_ref,
                  bf2_ref, out_ref, pool_acc, cnt_acc, news_acc):
    j = pl.program_id(0)

    @pl.when(j == 0)
    def _():
        pool_acc[...] = jnp.zeros_like(pool_acc)
        cnt_acc[...] = jnp.zeros_like(cnt_acc)
        news_acc[...] = jnp.zeros_like(news_acc)

    a = jnp.concatenate([alo_ref[...], ahi_ref[...]], axis=1)
    h = _selu(a * dinv_ref[...] + b2_ref[...])

    gvec = lax.broadcasted_iota(jnp.int32, (1, G), 1)
    m = (batch_ref[...] == gvec).astype(f32)
    dn = (((0,), (0,)), ((), ()))
    pool_acc[...] += lax.dot_general(m, h, dn, preferred_element_type=f32)
    cnt_acc[...] += lax.dot_general(
        m, jnp.ones((RB, 1), f32), dn, preferred_element_type=f32)

    rowid = (lax.broadcasted_iota(jnp.int32, (RB, 1), 0) + j * RB).astype(f32)
    fsel = (rowid == fidx_ref[...]).astype(f32)
    news_acc[...] += lax.dot_general(fsel, x_ref[...], dn,
                                     preferred_element_type=f32)

    @pl.when(j == NSTEP - 1)
    def _():
        cnt = jnp.maximum(cnt_acc[...], 1.0)
        pooled = _selu(pool_acc[...] / cnt)
        news = jnp.maximum(
            jnp.dot(news_acc[...], wf0_ref[...], preferred_element_type=f32)
            + bf0_ref[...], 0.0)
        z = jnp.concatenate([pooled, news], axis=1)
        z = _selu(jnp.dot(z, wf1_ref[...], preferred_element_type=f32)
                  + bf1_ref[...])
        z = jnp.dot(z, wf2_ref[...], preferred_element_type=f32) + bf2_ref[...]
        zmax = jnp.max(z, axis=-1, keepdims=True)
        lse = zmax + jnp.log(jnp.sum(jnp.exp(z - zmax), axis=-1, keepdims=True))
        out_ref[...] = z - lse


def _tc_head(alo, ahi, dinv, b2, batch2, x, fidx, wf0, bf0, wf1, bf1,
             wf2, bf2):
    full = lambda shape: pl.BlockSpec(shape, lambda j: tuple(0 for _ in shape))
    return pl.pallas_call(
        _tc_head_body,
        grid=(NSTEP,),
        in_specs=[
            pl.BlockSpec((RB, HALF), lambda j: (j, 0)),
            pl.BlockSpec((RB, HALF), lambda j: (j, 0)),
            pl.BlockSpec((RB, 1), lambda j: (j, 0)),
            full((1, H2)),
            pl.BlockSpec((RB, 1), lambda j: (j, 0)),
            pl.BlockSpec((RB, F_IN), lambda j: (j, 0)),
            full((1, G)),
            full((F_IN, H2)),
            full((1, H2)),
            full((2 * H2, 128)),
            full((1, 128)),
            full((128, C)),
            full((1, C)),
        ],
        out_specs=full((G, C)),
        out_shape=jax.ShapeDtypeStruct((G, C), f32),
        scratch_shapes=[
            pltpu.VMEM((G, H2), f32),
            pltpu.VMEM((G, 1), f32),
            pltpu.VMEM((G, F_IN), f32),
        ],
    )(alo, ahi, dinv, b2, batch2, x, fidx, wf0, bf0, wf1, bf1, wf2, bf2)


def kernel(x, edge_index, batch, W1, b1, W2, b2, Wf0, bf0, Wf1, bf1, Wf2, bf2):
    srcp = jnp.pad(edge_index[0], (0, EP - E), constant_values=NP - 1)
    dstp = jnp.pad(edge_index[1], (0, EP - E), constant_values=NP - 1)
    src2 = srcp.reshape(ECHS, SPC)
    dst2 = dstp.reshape(ECHS, SPC)
    dstd = dstp.reshape(ECH, CHUNK)
    x = jnp.pad(x, ((0, NP - N), (0, 0)))
    batch2 = jnp.pad(batch, (0, NP - N), constant_values=G).reshape(NP, 1)

    sc_deg1, sc_deg2, sc_spmm = _sc_kernels()
    deg = sc_deg2(sc_deg1(dstd)).reshape(NP, 1)
    hlo, hhi, dinv, fidx = _tc_pre(deg, x, W1, batch2)
    alo, ahi = sc_spmm(hlo, hhi, src2, dst2)
    h2lo, h2hi = _tc_mid(alo, ahi, dinv, b1.reshape(1, H2), W2)
    a2lo, a2hi = sc_spmm(h2lo, h2hi, src2, dst2)
    return _tc_head(a2lo, a2hi, dinv, b2.reshape(1, H2), batch2, x, fidx,
                    Wf0, bf0.reshape(1, H2), Wf1, bf1.reshape(1, 128),
                    Wf2, bf2.reshape(1, C))


# speedup vs baseline: 1.1494x; 1.1494x over previous
import functools

import jax
import jax.numpy as jnp
from jax import lax
from jax.experimental import pallas as pl
from jax.experimental.pallas import tpu as pltpu
from jax.experimental.pallas import tpu_sc as plsc

N = 10000
NP = 10240
E = 320000
EP = 327680
F_IN = 128
H2 = 256
HALF = 128
C = 2
G = 128

NC = 2
NS = 16
CHUNK = 128
ECH = EP // CHUNK
SPC = 64
ECHS = EP // SPC

NCH_DEG = ECH // (NC * NS)
NCH_SP = ECHS // NS
NSTG = 8
SCH = NCH_SP // NSTG
NBUF = 4
KPF = 2
RPT = NP // NS
CPT = NP // NS

f32 = jnp.float32


@functools.lru_cache(maxsize=None)
def _sc_kernels():
    mesh = plsc.VectorSubcoreMesh(
        core_axis_name="c", subcore_axis_name="s",
        num_cores=NC, num_subcores=NS)

    @functools.partial(
        pl.kernel,
        out_type=jax.ShapeDtypeStruct((NC, 1, NP), f32),
        mesh=mesh,
        scratch_types=[
            pltpu.VMEM((NCH_DEG, CHUNK), jnp.int32),
            pltpu.VMEM((CHUNK,), f32),
            pltpu.VMEM((CPT,), f32),
            pltpu.VMEM_SHARED((NP,), f32),
        ],
    )
    def sc_deg1(dst_hbm, part_hbm, idx_v, ones_v, zbuf, acc_sh):
        c = lax.axis_index("c")
        s = lax.axis_index("s")
        wid = c * NS + s

        one16 = jnp.ones((16,), f32)
        zero16 = jnp.zeros((16,), f32)

        def fill_ones(i, _):
            ones_v[pl.ds(i * 16, 16)] = one16
            return 0

        lax.fori_loop(0, CHUNK // 16, fill_ones, 0)

        def fill_zero(i, _):
            zbuf[pl.ds(i * 16, 16)] = zero16
            return 0

        lax.fori_loop(0, CPT // 16, fill_zero, 0)

        pltpu.sync_copy(zbuf, acc_sh.at[pl.ds(s * CPT, CPT)])
        pltpu.sync_copy(dst_hbm.at[pl.ds(wid * NCH_DEG, NCH_DEG)], idx_v)
        plsc.subcore_barrier()

        def body(j, _):
            pltpu.sync_copy(ones_v, acc_sh.at[idx_v.at[j]], add=True)
            return 0

        lax.fori_loop(0, NCH_DEG, body, 0)
        plsc.subcore_barrier()
        pltpu.sync_copy(acc_sh.at[pl.ds(s * CPT, CPT)],
                        part_hbm.at[c, 0, pl.ds(s * CPT, CPT)])

    @functools.partial(
        pl.kernel,
        out_type=jax.ShapeDtypeStruct((NP,), f32),
        mesh=mesh,
        scratch_types=[
            pltpu.VMEM((NC, CPT), f32),
            pltpu.VMEM((CPT,), f32),
        ],
    )
    def sc_deg2(part_hbm, deg_hbm, pbuf, abuf):
        c = lax.axis_index("c")
        s = lax.axis_index("s")

        @pl.when(c == 0)
        def _():
            pltpu.sync_copy(
                part_hbm.at[:, 0, pl.ds(s * CPT, CPT)], pbuf)

            def red(k, _):
                v = pbuf[0, pl.ds(k * 16, 16)]
                for i in range(1, NC):
                    v = v + pbuf[i, pl.ds(k * 16, 16)]
                abuf[pl.ds(k * 16, 16)] = v
                return 0

            lax.fori_loop(0, CPT // 16, red, 0)
            pltpu.sync_copy(abuf, deg_hbm.at[pl.ds(s * CPT, CPT)])

    @functools.partial(
        pl.kernel,
        out_type=(
            jax.ShapeDtypeStruct((NP, HALF), f32),
            jax.ShapeDtypeStruct((NP, HALF), f32),
        ),
        mesh=mesh,
        scratch_types=(
            [pltpu.VMEM((SCH, SPC), jnp.int32)] * 2
            + [pltpu.VMEM((SPC, HALF), f32)] * NBUF
            + [pltpu.VMEM_SHARED((NP, HALF), f32)]
            + [pltpu.SemaphoreType.DMA] * (2 * NBUF)
        ),
    )
    def sc_spmm(hlo_hbm, hhi_hbm, src_hbm, dst_hbm, alo_hbm, ahi_hbm,
                *scratch):
        src_v, dst_v = scratch[0], scratch[1]
        rows = scratch[2:2 + NBUF]
        acc_sh = scratch[2 + NBUF]
        gsems = scratch[3 + NBUF:3 + 2 * NBUF]
        ssems = scratch[3 + 2 * NBUF:3 + 3 * NBUF]
        c = lax.axis_index("c")
        s = lax.axis_index("s")

        def do_half(h_ref, agg_ref):
            pltpu.sync_copy(
                h_ref.at[pl.ds(s * RPT, RPT)],
                acc_sh.at[pl.ds(s * RPT, RPT)],
            )
            plsc.subcore_barrier()

            def gather(jj, b):
                pltpu.async_copy(h_ref.at[src_v.at[jj]], rows[b], gsems[b])

            def gwait(b):
                pltpu.make_async_copy(
                    h_ref.at[pl.ds(0, SPC)], rows[b], gsems[b]).wait()

            def scatter(jj, b):
                pltpu.async_copy(
                    rows[b], acc_sh.at[dst_v.at[jj]], ssems[b], add=True)

            def swait(b):
                pltpu.make_async_copy(
                    rows[b], acc_sh.at[pl.ds(0, SPC)], ssems[b]).wait()

            def stage(q):
                base = pl.multiple_of(s * NCH_SP + q * SCH, 8)
                pltpu.sync_copy(src_hbm.at[pl.ds(base, SCH)], src_v)
                pltpu.sync_copy(dst_hbm.at[pl.ds(base, SCH)], dst_v)
                for t in range(KPF):
                    gather(t, t)

                def lbody(j):
                    for bi in range(NBUF):
                        jj = j + bi
                        b = bi
                        nb = (bi + KPF) % NBUF

                        @pl.when(jj >= NBUF - KPF)
                        def _():
                            swait(nb)

                        @pl.when(jj + KPF < SCH)
                        def _():
                            gather(jj + KPF, nb)

                        gwait(b)
                        scatter(jj, b)

                pl.loop(0, SCH, step=NBUF)(lbody)
                for m in range(SCH - (NBUF - KPF), SCH):
                    swait(m % NBUF)

            pl.loop(0, NSTG)(stage)
            plsc.subcore_barrier()

            pltpu.sync_copy(
                acc_sh.at[pl.ds(s * RPT, RPT)],
                agg_ref.at[pl.ds(s * RPT, RPT)],
            )
            plsc.subcore_barrier()

        @pl.when(c == 0)
        def _():
            do_half(hlo_hbm, alo_hbm)

        @pl.when(c == 1)
        def _():
            do_half(hhi_hbm, ahi_hbm)

    return sc_deg1, sc_deg2, sc_spmm


_SELU_ALPHA = 1.6732632423543772
_SELU_SCALE = 1.0507009873554805


def _selu(x):
    return _SELU_SCALE * jnp.where(x > 0, x, _SELU_ALPHA * (jnp.exp(x) - 1.0))


RB = 2560
NSTEP = NP // RB


def _tc_pre_body(deg_ref, x_ref, w1_ref, batch_ref,
                 hlo_ref, hhi_ref, dinv_ref, fidx_ref, cnt_acc):
    j = pl.program_id(0)

    @pl.when(j == 0)
    def _():
        cnt_acc[...] = jnp.zeros_like(cnt_acc)

    deg = deg_ref[...] + 1.0
    dinv = lax.rsqrt(deg)
    dinv_ref[...] = dinv
    h = jnp.dot(x_ref[...], w1_ref[...], preferred_element_type=f32) * dinv
    hlo_ref[...] = h[:, :HALF]
    hhi_ref[...] = h[:, HALF:]

    gvec = lax.broadcasted_iota(jnp.int32, (1, G), 1)
    lt = (batch_ref[...] < gvec).astype(f32)
    cnt_acc[...] += jnp.sum(lt, axis=0, keepdims=True)

    @pl.when(j == NSTEP - 1)
    def _():
        fidx_ref[...] = jnp.minimum(cnt_acc[...], float(N - 1))


def _tc_pre(deg, x, w1, batch2):
    return pl.pallas_call(
        _tc_pre_body,
        grid=(NSTEP,),
        in_specs=[
            pl.BlockSpec((RB, 1), lambda j: (j, 0)),
            pl.BlockSpec((RB, F_IN), lambda j: (j, 0)),
            pl.BlockSpec((F_IN, H2), lambda j: (0, 0)),
            pl.BlockSpec((RB, 1), lambda j: (j, 0)),
        ],
        out_specs=[
            pl.BlockSpec((RB, HALF), lambda j: (j, 0)),
            pl.BlockSpec((RB, HALF), lambda j: (j, 0)),
            pl.BlockSpec((RB, 1), lambda j: (j, 0)),
            pl.BlockSpec((1, G), lambda j: (0, 0)),
        ],
        out_shape=[
            jax.ShapeDtypeStruct((NP, HALF), f32),
            jax.ShapeDtypeStruct((NP, HALF), f32),
            jax.ShapeDtypeStruct((NP, 1), f32),
            jax.ShapeDtypeStruct((1, G), f32),
        ],
        scratch_shapes=[pltpu.VMEM((1, G), f32)],
    )(deg, x, w1, batch2)


def _tc_mid_body(alo_ref, ahi_ref, dinv_ref, b1_ref, w2_ref,
                 hlo_ref, hhi_ref):
    dinv = dinv_ref[...]
    a = jnp.concatenate([alo_ref[...], ahi_ref[...]], axis=1)
    h1 = _selu(a * dinv + b1_ref[...])
    h2 = jnp.dot(h1, w2_ref[...], preferred_element_type=f32) * dinv
    hlo_ref[...] = h2[:, :HALF]
    hhi_ref[...] = h2[:, HALF:]


def _tc_mid(alo, ahi, dinv, b1, w2):
    return pl.pallas_call(
        _tc_mid_body,
        grid=(NSTEP,),
        in_specs=[
            pl.BlockSpec((RB, HALF), lambda j: (j, 0)),
            pl.BlockSpec((RB, HALF), lambda j: (j, 0)),
            pl.BlockSpec((RB, 1), lambda j: (j, 0)),
            pl.BlockSpec((1, H2), lambda j: (0, 0)),
            pl.BlockSpec((H2, H2), lambda j: (0, 0)),
        ],
        out_specs=[
            pl.BlockSpec((RB, HALF), lambda j: (j, 0)),
            pl.BlockSpec((RB, HALF), lambda j: (j, 0)),
        ],
        out_shape=[
            jax.ShapeDtypeStruct((NP, HALF), f32),
            jax.ShapeDtypeStruct((NP, HALF), f32),
        ],
    )(alo, ahi, dinv, b1, w2)


def _tc_head_body(alo_ref, ahi_ref, dinv_ref, b2_ref, batch_ref, x_ref,
                  fidx_ref, wf0_ref, bf0_ref, wf1_ref, bf1_ref, wf2_ref,
                  bf2_ref, out_ref, pool_acc, cnt_acc, news_acc):
    j = pl.program_id(0)

    @pl.when(j == 0)
    def _():
        pool_acc[...] = jnp.zeros_like(pool_acc)
        cnt_acc[...] = jnp.zeros_like(cnt_acc)
        news_acc[...] = jnp.zeros_like(news_acc)

    a = jnp.concatenate([alo_ref[...], ahi_ref[...]], axis=1)
    h = _selu(a * dinv_ref[...] + b2_ref[...])

    gvec = lax.broadcasted_iota(jnp.int32, (1, G), 1)
    m = (batch_ref[...] == gvec).astype(f32)
    dn = (((0,), (0,)), ((), ()))
    pool_acc[...] += lax.dot_general(m, h, dn, preferred_element_type=f32)
    cnt_acc[...] += lax.dot_general(
        m, jnp.ones((RB, 1), f32), dn, preferred_element_type=f32)

    rowid = (lax.broadcasted_iota(jnp.int32, (RB, 1), 0) + j * RB).astype(f32)
    fsel = (rowid == fidx_ref[...]).astype(f32)
    news_acc[...] += lax.dot_general(fsel, x_ref[...], dn,
                                     preferred_element_type=f32)

    @pl.when(j == NSTEP - 1)
    def _():
        cnt = jnp.maximum(cnt_acc[...], 1.0)
        pooled = _selu(pool_acc[...] / cnt)
        news = jnp.maximum(
            jnp.dot(news_acc[...], wf0_ref[...], preferred_element_type=f32)
            + bf0_ref[...], 0.0)
        z = jnp.concatenate([pooled, news], axis=1)
        z = _selu(jnp.dot(z, wf1_ref[...], preferred_element_type=f32)
                  + bf1_ref[...])
        z = jnp.dot(z, wf2_ref[...], preferred_element_type=f32) + bf2_ref[...]
        zmax = jnp.max(z, axis=-1, keepdims=True)
        lse = zmax + jnp.log(jnp.sum(jnp.exp(z - zmax), axis=-1, keepdims=True))
        out_ref[...] = z - lse


def _tc_head(alo, ahi, dinv, b2, batch2, x, fidx, wf0, bf0, wf1, bf1,
             wf2, bf2):
    full = lambda shape: pl.BlockSpec(shape, lambda j: tuple(0 for _ in shape))
    return pl.pallas_call(
        _tc_head_body,
        grid=(NSTEP,),
        in_specs=[
            pl.BlockSpec((RB, HALF), lambda j: (j, 0)),
            pl.BlockSpec((RB, HALF), lambda j: (j, 0)),
            pl.BlockSpec((RB, 1), lambda j: (j, 0)),
            full((1, H2)),
            pl.BlockSpec((RB, 1), lambda j: (j, 0)),
            pl.BlockSpec((RB, F_IN), lambda j: (j, 0)),
            full((1, G)),
            full((F_IN, H2)),
            full((1, H2)),
            full((2 * H2, 128)),
            full((1, 128)),
            full((128, C)),
            full((1, C)),
        ],
        out_specs=full((G, C)),
        out_shape=jax.ShapeDtypeStruct((G, C), f32),
        scratch_shapes=[
            pltpu.VMEM((G, H2), f32),
            pltpu.VMEM((G, 1), f32),
            pltpu.VMEM((G, F_IN), f32),
        ],
    )(alo, ahi, dinv, b2, batch2, x, fidx, wf0, bf0, wf1, bf1, wf2, bf2)


def kernel(x, edge_index, batch, W1, b1, W2, b2, Wf0, bf0, Wf1, bf1, Wf2, bf2):
    srcp = jnp.pad(edge_index[0], (0, EP - E), constant_values=NP - 1)
    dstp = jnp.pad(edge_index[1], (0, EP - E), constant_values=NP - 1)
    src2 = srcp.reshape(ECHS, SPC)
    dst2 = dstp.reshape(ECHS, SPC)
    dstd = dstp.reshape(ECH, CHUNK)
    x = jnp.pad(x, ((0, NP - N), (0, 0)))
    batch2 = jnp.pad(batch, (0, NP - N), constant_values=G).reshape(NP, 1)

    sc_deg1, sc_deg2, sc_spmm = _sc_kernels()
    deg = sc_deg2(sc_deg1(dstd)).reshape(NP, 1)
    hlo, hhi, dinv, fidx = _tc_pre(deg, x, W1, batch2)
    alo, ahi = sc_spmm(hlo, hhi, src2, dst2)
    h2lo, h2hi = _tc_mid(alo, ahi, dinv, b1.reshape(1, H2), W2)
    a2lo, a2hi = sc_spmm(h2lo, h2hi, src2, dst2)
    return _tc_head(a2lo, a2hi, dinv, b2.reshape(1, H2), batch2, x, fidx,
                    Wf0, bf0.reshape(1, H2), Wf1, bf1.reshape(1, 128),
                    Wf2, bf2.reshape(1, C))


# speedup vs baseline: 1.1619x; 1.0109x over previous
import functools

import jax
import jax.numpy as jnp
from jax import lax
from jax.experimental import pallas as pl
from jax.experimental.pallas import tpu as pltpu
from jax.experimental.pallas import tpu_sc as plsc

N = 10000
NP = 10240
E = 320000
EP = 327680
F_IN = 128
H2 = 256
HALF = 128
C = 2
G = 128

NC = 2
NS = 16
CHUNK = 128
ECH = EP // CHUNK
SPC = 64
ECHS = EP // SPC

NCH_DEG = ECH // (NC * NS)
NCH_SP = ECHS // NS
NSTG = 8
SCH = NCH_SP // NSTG
NBUF = 4
KPF = 3
RPT = NP // NS
CPT = NP // NS

f32 = jnp.float32


@functools.lru_cache(maxsize=None)
def _sc_kernels():
    mesh = plsc.VectorSubcoreMesh(
        core_axis_name="c", subcore_axis_name="s",
        num_cores=NC, num_subcores=NS)

    @functools.partial(
        pl.kernel,
        out_type=jax.ShapeDtypeStruct((NC, 1, NP), f32),
        mesh=mesh,
        scratch_types=[
            pltpu.VMEM((NCH_DEG, CHUNK), jnp.int32),
            pltpu.VMEM((CHUNK,), f32),
            pltpu.VMEM((CPT,), f32),
            pltpu.VMEM_SHARED((NP,), f32),
        ],
    )
    def sc_deg1(dst_hbm, part_hbm, idx_v, ones_v, zbuf, acc_sh):
        c = lax.axis_index("c")
        s = lax.axis_index("s")
        wid = c * NS + s

        one16 = jnp.ones((16,), f32)
        zero16 = jnp.zeros((16,), f32)

        def fill_ones(i, _):
            ones_v[pl.ds(i * 16, 16)] = one16
            return 0

        lax.fori_loop(0, CHUNK // 16, fill_ones, 0)

        def fill_zero(i, _):
            zbuf[pl.ds(i * 16, 16)] = zero16
            return 0

        lax.fori_loop(0, CPT // 16, fill_zero, 0)

        pltpu.sync_copy(zbuf, acc_sh.at[pl.ds(s * CPT, CPT)])
        pltpu.sync_copy(dst_hbm.at[pl.ds(wid * NCH_DEG, NCH_DEG)], idx_v)
        plsc.subcore_barrier()

        def body(j, _):
            pltpu.sync_copy(ones_v, acc_sh.at[idx_v.at[j]], add=True)
            return 0

        lax.fori_loop(0, NCH_DEG, body, 0)
        plsc.subcore_barrier()
        pltpu.sync_copy(acc_sh.at[pl.ds(s * CPT, CPT)],
                        part_hbm.at[c, 0, pl.ds(s * CPT, CPT)])

    @functools.partial(
        pl.kernel,
        out_type=jax.ShapeDtypeStruct((NP,), f32),
        mesh=mesh,
        scratch_types=[
            pltpu.VMEM((NC, CPT), f32),
            pltpu.VMEM((CPT,), f32),
        ],
    )
    def sc_deg2(part_hbm, deg_hbm, pbuf, abuf):
        c = lax.axis_index("c")
        s = lax.axis_index("s")

        @pl.when(c == 0)
        def _():
            pltpu.sync_copy(
                part_hbm.at[:, 0, pl.ds(s * CPT, CPT)], pbuf)

            def red(k, _):
                v = pbuf[0, pl.ds(k * 16, 16)]
                for i in range(1, NC):
                    v = v + pbuf[i, pl.ds(k * 16, 16)]
                abuf[pl.ds(k * 16, 16)] = v
                return 0

            lax.fori_loop(0, CPT // 16, red, 0)
            pltpu.sync_copy(abuf, deg_hbm.at[pl.ds(s * CPT, CPT)])

    @functools.partial(
        pl.kernel,
        out_type=(
            jax.ShapeDtypeStruct((NP, HALF), f32),
            jax.ShapeDtypeStruct((NP, HALF), f32),
        ),
        mesh=mesh,
        scratch_types=(
            [pltpu.VMEM((SCH, SPC), jnp.int32)] * 2
            + [pltpu.VMEM((SPC, HALF), f32)] * NBUF
            + [pltpu.VMEM_SHARED((NP, HALF), f32)]
            + [pltpu.SemaphoreType.DMA] * (2 * NBUF)
        ),
    )
    def sc_spmm(hlo_hbm, hhi_hbm, src_hbm, dst_hbm, alo_hbm, ahi_hbm,
                *scratch):
        src_v, dst_v = scratch[0], scratch[1]
        rows = scratch[2:2 + NBUF]
        acc_sh = scratch[2 + NBUF]
        gsems = scratch[3 + NBUF:3 + 2 * NBUF]
        ssems = scratch[3 + 2 * NBUF:3 + 3 * NBUF]
        c = lax.axis_index("c")
        s = lax.axis_index("s")

        def do_half(h_ref, agg_ref):
            pltpu.sync_copy(
                h_ref.at[pl.ds(s * RPT, RPT)],
                acc_sh.at[pl.ds(s * RPT, RPT)],
            )
            plsc.subcore_barrier()

            def gather(jj, b):
                pltpu.async_copy(h_ref.at[src_v.at[jj]], rows[b], gsems[b])

            def gwait(b):
                pltpu.make_async_copy(
                    h_ref.at[pl.ds(0, SPC)], rows[b], gsems[b]).wait()

            def scatter(jj, b):
                pltpu.async_copy(
                    rows[b], acc_sh.at[dst_v.at[jj]], ssems[b], add=True)

            def swait(b):
                pltpu.make_async_copy(
                    rows[b], acc_sh.at[pl.ds(0, SPC)], ssems[b]).wait()

            def stage(q):
                base = pl.multiple_of(s * NCH_SP + q * SCH, 8)
                pltpu.sync_copy(src_hbm.at[pl.ds(base, SCH)], src_v)
                pltpu.sync_copy(dst_hbm.at[pl.ds(base, SCH)], dst_v)
                for t in range(KPF):
                    gather(t, t)

                def lbody(j):
                    for bi in range(NBUF):
                        jj = j + bi
                        b = bi
                        nb = (bi + KPF) % NBUF

                        @pl.when(jj >= NBUF - KPF)
                        def _():
                            swait(nb)

                        @pl.when(jj + KPF < SCH)
                        def _():
                            gather(jj + KPF, nb)

                        gwait(b)
                        scatter(jj, b)

                pl.loop(0, SCH, step=NBUF)(lbody)
                for m in range(SCH - (NBUF - KPF), SCH):
                    swait(m % NBUF)

            pl.loop(0, NSTG)(stage)
            plsc.subcore_barrier()

            pltpu.sync_copy(
                acc_sh.at[pl.ds(s * RPT, RPT)],
                agg_ref.at[pl.ds(s * RPT, RPT)],
            )
            plsc.subcore_barrier()

        @pl.when(c == 0)
        def _():
            do_half(hlo_hbm, alo_hbm)

        @pl.when(c == 1)
        def _():
            do_half(hhi_hbm, ahi_hbm)

    return sc_deg1, sc_deg2, sc_spmm


_SELU_ALPHA = 1.6732632423543772
_SELU_SCALE = 1.0507009873554805


def _selu(x):
    return _SELU_SCALE * jnp.where(x > 0, x, _SELU_ALPHA * (jnp.exp(x) - 1.0))


RB = 2560
NSTEP = NP // RB


def _tc_pre_body(deg_ref, x_ref, w1_ref, batch_ref,
                 hlo_ref, hhi_ref, dinv_ref, fidx_ref, cnt_acc):
    j = pl.program_id(0)

    @pl.when(j == 0)
    def _():
        cnt_acc[...] = jnp.zeros_like(cnt_acc)

    deg = deg_ref[...] + 1.0
    dinv = lax.rsqrt(deg)
    dinv_ref[...] = dinv
    h = jnp.dot(x_ref[...], w1_ref[...], preferred_element_type=f32) * dinv
    hlo_ref[...] = h[:, :HALF]
    hhi_ref[...] = h[:, HALF:]

    gvec = lax.broadcasted_iota(jnp.int32, (1, G), 1)
    lt = (batch_ref[...] < gvec).astype(f32)
    cnt_acc[...] += jnp.sum(lt, axis=0, keepdims=True)

    @pl.when(j == NSTEP - 1)
    def _():
        fidx_ref[...] = jnp.minimum(cnt_acc[...], float(N - 1))


def _tc_pre(deg, x, w1, batch2):
    return pl.pallas_call(
        _tc_pre_body,
        grid=(NSTEP,),
        in_specs=[
            pl.BlockSpec((RB, 1), lambda j: (j, 0)),
            pl.BlockSpec((RB, F_IN), lambda j: (j, 0)),
            pl.BlockSpec((F_IN, H2), lambda j: (0, 0)),
            pl.BlockSpec((RB, 1), lambda j: (j, 0)),
        ],
        out_specs=[
            pl.BlockSpec((RB, HALF), lambda j: (j, 0)),
            pl.BlockSpec((RB, HALF), lambda j: (j, 0)),
            pl.BlockSpec((RB, 1), lambda j: (j, 0)),
            pl.BlockSpec((1, G), lambda j: (0, 0)),
        ],
        out_shape=[
            jax.ShapeDtypeStruct((NP, HALF), f32),
            jax.ShapeDtypeStruct((NP, HALF), f32),
            jax.ShapeDtypeStruct((NP, 1), f32),
            jax.ShapeDtypeStruct((1, G), f32),
        ],
        scratch_shapes=[pltpu.VMEM((1, G), f32)],
    )(deg, x, w1, batch2)


def _tc_mid_body(alo_ref, ahi_ref, dinv_ref, b1_ref, w2_ref,
                 hlo_ref, hhi_ref):
    dinv = dinv_ref[...]
    a = jnp.concatenate([alo_ref[...], ahi_ref[...]], axis=1)
    h1 = _selu(a * dinv + b1_ref[...])
    h2 = jnp.dot(h1, w2_ref[...], preferred_element_type=f32) * dinv
    hlo_ref[...] = h2[:, :HALF]
    hhi_ref[...] = h2[:, HALF:]


def _tc_mid(alo, ahi, dinv, b1, w2):
    return pl.pallas_call(
        _tc_mid_body,
        grid=(NSTEP,),
        in_specs=[
            pl.BlockSpec((RB, HALF), lambda j: (j, 0)),
            pl.BlockSpec((RB, HALF), lambda j: (j, 0)),
            pl.BlockSpec((RB, 1), lambda j: (j, 0)),
            pl.BlockSpec((1, H2), lambda j: (0, 0)),
            pl.BlockSpec((H2, H2), lambda j: (0, 0)),
        ],
        out_specs=[
            pl.BlockSpec((RB, HALF), lambda j: (j, 0)),
            pl.BlockSpec((RB, HALF), lambda j: (j, 0)),
        ],
        out_shape=[
            jax.ShapeDtypeStruct((NP, HALF), f32),
            jax.ShapeDtypeStruct((NP, HALF), f32),
        ],
    )(alo, ahi, dinv, b1, w2)


def _tc_head_body(alo_ref, ahi_ref, dinv_ref, b2_ref, batch_ref, x_ref,
                  fidx_ref, wf0_ref, bf0_ref, wf1_ref, bf1_ref, wf2_ref,
                  bf2_ref, out_ref, pool_acc, cnt_acc, news_acc):
    j = pl.program_id(0)

    @pl.when(j == 0)
    def _():
        pool_acc[...] = jnp.zeros_like(pool_acc)
        cnt_acc[...] = jnp.zeros_like(cnt_acc)
        news_acc[...] = jnp.zeros_like(news_acc)

    a = jnp.concatenate([alo_ref[...], ahi_ref[...]], axis=1)
    h = _selu(a * dinv_ref[...] + b2_ref[...])

    gvec = lax.broadcasted_iota(jnp.int32, (1, G), 1)
    m = (batch_ref[...] == gvec).astype(f32)
    dn = (((0,), (0,)), ((), ()))
    pool_acc[...] += lax.dot_general(m, h, dn, preferred_element_type=f32)
    cnt_acc[...] += lax.dot_general(
        m, jnp.ones((RB, 1), f32), dn, preferred_element_type=f32)

    rowid = (lax.broadcasted_iota(jnp.int32, (RB, 1), 0) + j * RB).astype(f32)
    fsel = (rowid == fidx_ref[...]).astype(f32)
    news_acc[...] += lax.dot_general(fsel, x_ref[...], dn,
                                     preferred_element_type=f32)

    @pl.when(j == NSTEP - 1)
    def _():
        cnt = jnp.maximum(cnt_acc[...], 1.0)
        pooled = _selu(pool_acc[...] / cnt)
        news = jnp.maximum(
            jnp.dot(news_acc[...], wf0_ref[...], preferred_element_type=f32)
            + bf0_ref[...], 0.0)
        z = jnp.concatenate([pooled, news], axis=1)
        z = _selu(jnp.dot(z, wf1_ref[...], preferred_element_type=f32)
                  + bf1_ref[...])
        z = jnp.dot(z, wf2_ref[...], preferred_element_type=f32) + bf2_ref[...]
        zmax = jnp.max(z, axis=-1, keepdims=True)
        lse = zmax + jnp.log(jnp.sum(jnp.exp(z - zmax), axis=-1, keepdims=True))
        out_ref[...] = z - lse


def _tc_head(alo, ahi, dinv, b2, batch2, x, fidx, wf0, bf0, wf1, bf1,
             wf2, bf2):
    full = lambda shape: pl.BlockSpec(shape, lambda j: tuple(0 for _ in shape))
    return pl.pallas_call(
        _tc_head_body,
        grid=(NSTEP,),
        in_specs=[
            pl.BlockSpec((RB, HALF), lambda j: (j, 0)),
            pl.BlockSpec((RB, HALF), lambda j: (j, 0)),
            pl.BlockSpec((RB, 1), lambda j: (j, 0)),
            full((1, H2)),
            pl.BlockSpec((RB, 1), lambda j: (j, 0)),
            pl.BlockSpec((RB, F_IN), lambda j: (j, 0)),
            full((1, G)),
            full((F_IN, H2)),
            full((1, H2)),
            full((2 * H2, 128)),
            full((1, 128)),
            full((128, C)),
            full((1, C)),
        ],
        out_specs=full((G, C)),
        out_shape=jax.ShapeDtypeStruct((G, C), f32),
        scratch_shapes=[
            pltpu.VMEM((G, H2), f32),
            pltpu.VMEM((G, 1), f32),
            pltpu.VMEM((G, F_IN), f32),
        ],
    )(alo, ahi, dinv, b2, batch2, x, fidx, wf0, bf0, wf1, bf1, wf2, bf2)


def kernel(x, edge_index, batch, W1, b1, W2, b2, Wf0, bf0, Wf1, bf1, Wf2, bf2):
    srcp = jnp.pad(edge_index[0], (0, EP - E), constant_values=NP - 1)
    dstp = jnp.pad(edge_index[1], (0, EP - E), constant_values=NP - 1)
    src2 = srcp.reshape(ECHS, SPC)
    dst2 = dstp.reshape(ECHS, SPC)
    dstd = dstp.reshape(ECH, CHUNK)
    x = jnp.pad(x, ((0, NP - N), (0, 0)))
    batch2 = jnp.pad(batch, (0, NP - N), constant_values=G).reshape(NP, 1)

    sc_deg1, sc_deg2, sc_spmm = _sc_kernels()
    deg = sc_deg2(sc_deg1(dstd)).reshape(NP, 1)
    hlo, hhi, dinv, fidx = _tc_pre(deg, x, W1, batch2)
    alo, ahi = sc_spmm(hlo, hhi, src2, dst2)
    h2lo, h2hi = _tc_mid(alo, ahi, dinv, b1.reshape(1, H2), W2)
    a2lo, a2hi = sc_spmm(h2lo, h2hi, src2, dst2)
    return _tc_head(a2lo, a2hi, dinv, b2.reshape(1, H2), batch2, x, fidx,
                    Wf0, bf0.reshape(1, H2), Wf1, bf1.reshape(1, 128),
                    Wf2, bf2.reshape(1, C))


# speedup vs baseline: 1.1637x; 1.0016x over previous
import functools

import jax
import jax.numpy as jnp
from jax import lax
from jax.experimental import pallas as pl
from jax.experimental.pallas import tpu as pltpu
from jax.experimental.pallas import tpu_sc as plsc

N = 10000
NP = 10240
E = 320000
EP = 327680
F_IN = 128
H2 = 256
HALF = 128
C = 2
G = 128

NC = 2
NS = 16
CHUNK = 128
ECH = EP // CHUNK
SPC = 64
ECHS = EP // SPC

NCH_DEG = ECH // (NC * NS)
NCH_SP = ECHS // NS
NSTG = 8
SCH = NCH_SP // NSTG
NBUF = 4
KPF = 3
RPT = NP // NS
CPT = NP // NS

f32 = jnp.float32


@functools.lru_cache(maxsize=None)
def _sc_kernels():
    mesh = plsc.VectorSubcoreMesh(
        core_axis_name="c", subcore_axis_name="s",
        num_cores=NC, num_subcores=NS)

    @functools.partial(
        pl.kernel,
        out_type=jax.ShapeDtypeStruct((NC, 1, NP), f32),
        mesh=mesh,
        scratch_types=[
            pltpu.VMEM((NCH_DEG, CHUNK), jnp.int32),
            pltpu.VMEM((CHUNK,), f32),
            pltpu.VMEM((CPT,), f32),
            pltpu.VMEM_SHARED((NP,), f32),
        ],
    )
    def sc_deg1(dst_hbm, part_hbm, idx_v, ones_v, zbuf, acc_sh):
        c = lax.axis_index("c")
        s = lax.axis_index("s")
        wid = c * NS + s

        one16 = jnp.ones((16,), f32)
        zero16 = jnp.zeros((16,), f32)

        def fill_ones(i, _):
            ones_v[pl.ds(i * 16, 16)] = one16
            return 0

        lax.fori_loop(0, CHUNK // 16, fill_ones, 0)

        def fill_zero(i, _):
            zbuf[pl.ds(i * 16, 16)] = zero16
            return 0

        lax.fori_loop(0, CPT // 16, fill_zero, 0)

        pltpu.sync_copy(zbuf, acc_sh.at[pl.ds(s * CPT, CPT)])
        pltpu.sync_copy(dst_hbm.at[pl.ds(wid * NCH_DEG, NCH_DEG)], idx_v)
        plsc.subcore_barrier()

        def body(j, _):
            pltpu.sync_copy(ones_v, acc_sh.at[idx_v.at[j]], add=True)
            return 0

        lax.fori_loop(0, NCH_DEG, body, 0)
        plsc.subcore_barrier()
        pltpu.sync_copy(acc_sh.at[pl.ds(s * CPT, CPT)],
                        part_hbm.at[c, 0, pl.ds(s * CPT, CPT)])

    @functools.partial(
        pl.kernel,
        out_type=jax.ShapeDtypeStruct((NP,), f32),
        mesh=mesh,
        scratch_types=[
            pltpu.VMEM((NC, CPT), f32),
            pltpu.VMEM((CPT,), f32),
        ],
    )
    def sc_deg2(part_hbm, deg_hbm, pbuf, abuf):
        c = lax.axis_index("c")
        s = lax.axis_index("s")

        @pl.when(c == 0)
        def _():
            pltpu.sync_copy(
                part_hbm.at[:, 0, pl.ds(s * CPT, CPT)], pbuf)

            def red(k, _):
                v = pbuf[0, pl.ds(k * 16, 16)]
                for i in range(1, NC):
                    v = v + pbuf[i, pl.ds(k * 16, 16)]
                abuf[pl.ds(k * 16, 16)] = v
                return 0

            lax.fori_loop(0, CPT // 16, red, 0)
            pltpu.sync_copy(abuf, deg_hbm.at[pl.ds(s * CPT, CPT)])

    @functools.partial(
        pl.kernel,
        out_type=(
            jax.ShapeDtypeStruct((NP, HALF), f32),
            jax.ShapeDtypeStruct((NP, HALF), f32),
        ),
        mesh=mesh,
        scratch_types=(
            [pltpu.VMEM((SCH, SPC), jnp.int32)] * 2
            + [pltpu.VMEM((SPC, HALF), f32)] * NBUF
            + [pltpu.VMEM_SHARED((NP, HALF), f32)]
            + [pltpu.SemaphoreType.DMA] * (2 * NBUF)
        ),
    )
    def sc_spmm(hlo_hbm, hhi_hbm, src_hbm, dst_hbm, alo_hbm, ahi_hbm,
                *scratch):
        src_v, dst_v = scratch[0], scratch[1]
        rows = scratch[2:2 + NBUF]
        acc_sh = scratch[2 + NBUF]
        gsems = scratch[3 + NBUF:3 + 2 * NBUF]
        ssems = scratch[3 + 2 * NBUF:3 + 3 * NBUF]
        c = lax.axis_index("c")
        s = lax.axis_index("s")

        def do_half(h_ref, agg_ref):
            pltpu.sync_copy(
                h_ref.at[pl.ds(s * RPT, RPT)],
                acc_sh.at[pl.ds(s * RPT, RPT)],
            )
            plsc.subcore_barrier()

            def gather(jj, b):
                pltpu.async_copy(h_ref.at[src_v.at[jj]], rows[b], gsems[b])

            def gwait(b):
                pltpu.make_async_copy(
                    h_ref.at[pl.ds(0, SPC)], rows[b], gsems[b]).wait()

            def scatter(jj, b):
                pltpu.async_copy(
                    rows[b], acc_sh.at[dst_v.at[jj]], ssems[b], add=True)

            def swait(b):
                pltpu.make_async_copy(
                    rows[b], acc_sh.at[pl.ds(0, SPC)], ssems[b]).wait()

            def stage(q):
                base = pl.multiple_of(s * NCH_SP + q * SCH, 8)
                pltpu.sync_copy(src_hbm.at[pl.ds(base, SCH)], src_v)
                pltpu.sync_copy(dst_hbm.at[pl.ds(base, SCH)], dst_v)
                for t in range(KPF):
                    gather(t, t)

                def lbody(j):
                    for bi in range(NBUF):
                        jj = j + bi
                        b = bi
                        nb = (bi + KPF) % NBUF

                        @pl.when(jj >= NBUF - KPF)
                        def _():
                            swait(nb)

                        @pl.when(jj + KPF < SCH)
                        def _():
                            gather(jj + KPF, nb)

                        gwait(b)
                        scatter(jj, b)

                pl.loop(0, SCH, step=NBUF)(lbody)
                for m in range(SCH - (NBUF - KPF), SCH):
                    swait(m % NBUF)

            pl.loop(0, NSTG)(stage)
            plsc.subcore_barrier()

            pltpu.sync_copy(
                acc_sh.at[pl.ds(s * RPT, RPT)],
                agg_ref.at[pl.ds(s * RPT, RPT)],
            )
            plsc.subcore_barrier()

        @pl.when(c == 0)
        def _():
            do_half(hlo_hbm, alo_hbm)

        @pl.when(c == 1)
        def _():
            do_half(hhi_hbm, ahi_hbm)

    return sc_deg1, sc_deg2, sc_spmm


_SELU_ALPHA = 1.6732632423543772
_SELU_SCALE = 1.0507009873554805


def _selu(x):
    return _SELU_SCALE * jnp.where(x > 0, x, _SELU_ALPHA * (jnp.exp(x) - 1.0))


RB = 5120
NSTEP = NP // RB


def _tc_pre_body(deg_ref, x_ref, w1_ref, batch_ref,
                 hlo_ref, hhi_ref, dinv_ref, fidx_ref, cnt_acc):
    j = pl.program_id(0)

    @pl.when(j == 0)
    def _():
        cnt_acc[...] = jnp.zeros_like(cnt_acc)

    deg = deg_ref[...] + 1.0
    dinv = lax.rsqrt(deg)
    dinv_ref[...] = dinv
    h = jnp.dot(x_ref[...], w1_ref[...], preferred_element_type=f32) * dinv
    hlo_ref[...] = h[:, :HALF]
    hhi_ref[...] = h[:, HALF:]

    gvec = lax.broadcasted_iota(jnp.int32, (1, G), 1)
    lt = (batch_ref[...] < gvec).astype(f32)
    cnt_acc[...] += jnp.sum(lt, axis=0, keepdims=True)

    @pl.when(j == NSTEP - 1)
    def _():
        fidx_ref[...] = jnp.minimum(cnt_acc[...], float(N - 1))


def _tc_pre(deg, x, w1, batch2):
    return pl.pallas_call(
        _tc_pre_body,
        grid=(NSTEP,),
        in_specs=[
            pl.BlockSpec((RB, 1), lambda j: (j, 0)),
            pl.BlockSpec((RB, F_IN), lambda j: (j, 0)),
            pl.BlockSpec((F_IN, H2), lambda j: (0, 0)),
            pl.BlockSpec((RB, 1), lambda j: (j, 0)),
        ],
        out_specs=[
            pl.BlockSpec((RB, HALF), lambda j: (j, 0)),
            pl.BlockSpec((RB, HALF), lambda j: (j, 0)),
            pl.BlockSpec((RB, 1), lambda j: (j, 0)),
            pl.BlockSpec((1, G), lambda j: (0, 0)),
        ],
        out_shape=[
            jax.ShapeDtypeStruct((NP, HALF), f32),
            jax.ShapeDtypeStruct((NP, HALF), f32),
            jax.ShapeDtypeStruct((NP, 1), f32),
            jax.ShapeDtypeStruct((1, G), f32),
        ],
        scratch_shapes=[pltpu.VMEM((1, G), f32)],
    )(deg, x, w1, batch2)


def _tc_mid_body(alo_ref, ahi_ref, dinv_ref, b1_ref, w2_ref,
                 hlo_ref, hhi_ref):
    dinv = dinv_ref[...]
    a = jnp.concatenate([alo_ref[...], ahi_ref[...]], axis=1)
    h1 = _selu(a * dinv + b1_ref[...])
    h2 = jnp.dot(h1, w2_ref[...], preferred_element_type=f32) * dinv
    hlo_ref[...] = h2[:, :HALF]
    hhi_ref[...] = h2[:, HALF:]


def _tc_mid(alo, ahi, dinv, b1, w2):
    return pl.pallas_call(
        _tc_mid_body,
        grid=(NSTEP,),
        in_specs=[
            pl.BlockSpec((RB, HALF), lambda j: (j, 0)),
            pl.BlockSpec((RB, HALF), lambda j: (j, 0)),
            pl.BlockSpec((RB, 1), lambda j: (j, 0)),
            pl.BlockSpec((1, H2), lambda j: (0, 0)),
            pl.BlockSpec((H2, H2), lambda j: (0, 0)),
        ],
        out_specs=[
            pl.BlockSpec((RB, HALF), lambda j: (j, 0)),
            pl.BlockSpec((RB, HALF), lambda j: (j, 0)),
        ],
        out_shape=[
            jax.ShapeDtypeStruct((NP, HALF), f32),
            jax.ShapeDtypeStruct((NP, HALF), f32),
        ],
    )(alo, ahi, dinv, b1, w2)


def _tc_head_body(alo_ref, ahi_ref, dinv_ref, b2_ref, batch_ref, x_ref,
                  fidx_ref, wf0_ref, bf0_ref, wf1_ref, bf1_ref, wf2_ref,
                  bf2_ref, out_ref, pool_acc, cnt_acc, news_acc):
    j = pl.program_id(0)

    @pl.when(j == 0)
    def _():
        pool_acc[...] = jnp.zeros_like(pool_acc)
        cnt_acc[...] = jnp.zeros_like(cnt_acc)
        news_acc[...] = jnp.zeros_like(news_acc)

    a = jnp.concatenate([alo_ref[...], ahi_ref[...]], axis=1)
    h = _selu(a * dinv_ref[...] + b2_ref[...])

    gvec = lax.broadcasted_iota(jnp.int32, (1, G), 1)
    m = (batch_ref[...] == gvec).astype(f32)
    dn = (((0,), (0,)), ((), ()))
    pool_acc[...] += lax.dot_general(m, h, dn, preferred_element_type=f32)
    cnt_acc[...] += lax.dot_general(
        m, jnp.ones((RB, 1), f32), dn, preferred_element_type=f32)

    rowid = (lax.broadcasted_iota(jnp.int32, (RB, 1), 0) + j * RB).astype(f32)
    fsel = (rowid == fidx_ref[...]).astype(f32)
    news_acc[...] += lax.dot_general(fsel, x_ref[...], dn,
                                     preferred_element_type=f32)

    @pl.when(j == NSTEP - 1)
    def _():
        cnt = jnp.maximum(cnt_acc[...], 1.0)
        pooled = _selu(pool_acc[...] / cnt)
        news = jnp.maximum(
            jnp.dot(news_acc[...], wf0_ref[...], preferred_element_type=f32)
            + bf0_ref[...], 0.0)
        z = jnp.concatenate([pooled, news], axis=1)
        z = _selu(jnp.dot(z, wf1_ref[...], preferred_element_type=f32)
                  + bf1_ref[...])
        z = jnp.dot(z, wf2_ref[...], preferred_element_type=f32) + bf2_ref[...]
        zmax = jnp.max(z, axis=-1, keepdims=True)
        lse = zmax + jnp.log(jnp.sum(jnp.exp(z - zmax), axis=-1, keepdims=True))
        out_ref[...] = z - lse


def _tc_head(alo, ahi, dinv, b2, batch2, x, fidx, wf0, bf0, wf1, bf1,
             wf2, bf2):
    full = lambda shape: pl.BlockSpec(shape, lambda j: tuple(0 for _ in shape))
    return pl.pallas_call(
        _tc_head_body,
        grid=(NSTEP,),
        in_specs=[
            pl.BlockSpec((RB, HALF), lambda j: (j, 0)),
            pl.BlockSpec((RB, HALF), lambda j: (j, 0)),
            pl.BlockSpec((RB, 1), lambda j: (j, 0)),
            full((1, H2)),
            pl.BlockSpec((RB, 1), lambda j: (j, 0)),
            pl.BlockSpec((RB, F_IN), lambda j: (j, 0)),
            full((1, G)),
            full((F_IN, H2)),
            full((1, H2)),
            full((2 * H2, 128)),
            full((1, 128)),
            full((128, C)),
            full((1, C)),
        ],
        out_specs=full((G, C)),
        out_shape=jax.ShapeDtypeStruct((G, C), f32),
        scratch_shapes=[
            pltpu.VMEM((G, H2), f32),
            pltpu.VMEM((G, 1), f32),
            pltpu.VMEM((G, F_IN), f32),
        ],
    )(alo, ahi, dinv, b2, batch2, x, fidx, wf0, bf0, wf1, bf1, wf2, bf2)


def kernel(x, edge_index, batch, W1, b1, W2, b2, Wf0, bf0, Wf1, bf1, Wf2, bf2):
    srcp = jnp.pad(edge_index[0], (0, EP - E), constant_values=NP - 1)
    dstp = jnp.pad(edge_index[1], (0, EP - E), constant_values=NP - 1)
    src2 = srcp.reshape(ECHS, SPC)
    dst2 = dstp.reshape(ECHS, SPC)
    dstd = dstp.reshape(ECH, CHUNK)
    x = jnp.pad(x, ((0, NP - N), (0, 0)))
    batch2 = jnp.pad(batch, (0, NP - N), constant_values=G).reshape(NP, 1)

    sc_deg1, sc_deg2, sc_spmm = _sc_kernels()
    deg = sc_deg2(sc_deg1(dstd)).reshape(NP, 1)
    hlo, hhi, dinv, fidx = _tc_pre(deg, x, W1, batch2)
    alo, ahi = sc_spmm(hlo, hhi, src2, dst2)
    h2lo, h2hi = _tc_mid(alo, ahi, dinv, b1.reshape(1, H2), W2)
    a2lo, a2hi = sc_spmm(h2lo, h2hi, src2, dst2)
    return _tc_head(a2lo, a2hi, dinv, b2.reshape(1, H2), batch2, x, fidx,
                    Wf0, bf0.reshape(1, H2), Wf1, bf1.reshape(1, 128),
                    Wf2, bf2.reshape(1, C))
